# Initial kernel scaffold; baseline (speedup 1.0000x reference)
import jax
import jax.numpy as jnp
from jax.experimental import pallas as pl


def kernel(v_emb, batch_idx, W, b, tokens_K, tokens_V):
    raise NotImplementedError("write your pallas kernel here")


# speedup vs baseline: 25.6953x; 25.6953x over previous
import math

import jax
import jax.numpy as jnp
from jax import lax
from jax.experimental import pallas as pl
from jax.experimental.pallas import tpu as pltpu

_B = 128
_TOPK = 8


def _body(x_ref, idx_ref, w_ref, b_ref, k_ref, v_ref, out_ref, acc_ref):
    i = pl.program_id(0)
    nb = pl.num_programs(0)
    blk, D = x_ref.shape
    T = k_ref.shape[0]
    scale = 1.0 / math.sqrt(D)

    @pl.when(i == 0)
    def _init():
        acc_ref[...] = jnp.zeros_like(acc_ref)

    M = lax.dot_general(w_ref[...], k_ref[...], (((0,), (1,)), ((), ())),
                        preferred_element_type=jnp.float32) * scale
    c = lax.dot_general(b_ref[...], k_ref[...], (((1,), (1,)), ((), ())),
                        preferred_element_type=jnp.float32) * scale
    s = jnp.dot(x_ref[...], M, preferred_element_type=jnp.float32) + c

    neg = jnp.float32(-jnp.inf)
    cur = s
    row_max = None
    thresh = None
    for it in range(_TOPK):
        m = jnp.max(cur, axis=-1, keepdims=True)
        if it == 0:
            row_max = m
        thresh = m
        if it != _TOPK - 1:
            cur = jnp.where(cur == m, neg, cur)

    w = jnp.where(s >= thresh, jnp.exp(s - row_max), jnp.float32(0.0))
    w = w / jnp.sum(w, axis=-1, keepdims=True)

    idx = idx_ref[0]
    rows = lax.broadcasted_iota(jnp.int32, (_B, blk), 0)
    oh = (rows == idx).astype(jnp.float32)
    colT = lax.broadcasted_iota(jnp.int32, (blk, T), 1)
    cnt_col = jnp.where(colT == 0, jnp.float32(1.0), jnp.float32(0.0))
    rhs = jnp.concatenate([w, cnt_col], axis=1)
    acc_ref[...] += lax.dot_general(oh, rhs, (((1,), (0,)), ((), ())),
                                    preferred_element_type=jnp.float32)

    @pl.when(i == nb - 1)
    def _fin():
        seg = acc_ref[:, :T]
        cnt = acc_ref[:, T:T + 1]
        pooled = jnp.dot(seg, v_ref[...],
                         preferred_element_type=jnp.float32)
        out_ref[...] = pooled / jnp.maximum(cnt, jnp.float32(1.0))


def kernel(v_emb, batch_idx, W, b, tokens_K, tokens_V):
    N, D = v_emb.shape
    T = tokens_K.shape[0]
    blk = 2000
    idx = batch_idx.astype(jnp.int32)
    Np = ((N + blk - 1) // blk) * blk
    if Np != N:
        v_emb = jnp.pad(v_emb, ((0, Np - N), (0, 0)))
        idx = jnp.pad(idx, (0, Np - N), constant_values=_B)
    nb = Np // blk
    idx3 = idx.reshape(nb, 1, blk)
    b2 = b.reshape(1, D)

    return pl.pallas_call(
        _body,
        grid=(nb,),
        in_specs=[
            pl.BlockSpec((blk, D), lambda i: (i, 0)),
            pl.BlockSpec((1, 1, blk), lambda i: (i, 0, 0)),
            pl.BlockSpec((D, D), lambda i: (0, 0)),
            pl.BlockSpec((1, D), lambda i: (0, 0)),
            pl.BlockSpec((T, D), lambda i: (0, 0)),
            pl.BlockSpec((T, D), lambda i: (0, 0)),
        ],
        out_specs=pl.BlockSpec((_B, D), lambda i: (0, 0)),
        out_shape=jax.ShapeDtypeStruct((_B, D), jnp.float32),
        scratch_shapes=[pltpu.VMEM((_B, 2 * T), jnp.float32)],
    )(v_emb, idx3, W, b2, tokens_K, tokens_V)


# speedup vs baseline: 39.8686x; 1.5516x over previous
import math

import jax
import jax.numpy as jnp
from jax import lax
from jax.experimental import pallas as pl
from jax.experimental.pallas import tpu as pltpu

_B = 128
_TOPK = 8
_SUB = 256


def _body(x_ref, idx_ref, w_ref, b_ref, k_ref, v_ref, out_ref, acc_ref):
    i = pl.program_id(0)
    nb = pl.num_programs(0)
    blk, D = x_ref.shape
    T = k_ref.shape[0]
    scale = 1.0 / math.sqrt(D)

    @pl.when(i == 0)
    def _init():
        acc_ref[...] = jnp.zeros_like(acc_ref)

    M = lax.dot_general(w_ref[...], k_ref[...], (((0,), (1,)), ((), ())),
                        preferred_element_type=jnp.float32) * scale
    c = lax.dot_general(b_ref[...], k_ref[...], (((1,), (1,)), ((), ())),
                        preferred_element_type=jnp.float32) * scale
    s = jnp.dot(x_ref[...], M, preferred_element_type=jnp.float32) + c
    sT = jnp.swapaxes(s, 0, 1)

    idx = idx_ref[0]
    rowsB = lax.broadcasted_iota(jnp.int32, (_B, _SUB), 0)
    ones_rows = jnp.ones((8, _SUB), jnp.float32)
    neg = jnp.float32(-jnp.inf)

    part = jnp.zeros((T + 8, _B), jnp.float32)
    for j in range(blk // _SUB):
        sj = sT[:, j * _SUB:(j + 1) * _SUB]
        cur = sj
        col_max = None
        thresh = None
        for it in range(_TOPK):
            m = jnp.max(cur, axis=0, keepdims=True)
            if it == 0:
                col_max = m
            thresh = m
            if it != _TOPK - 1:
                cur = jnp.where(cur == m, neg, cur)
        w = jnp.where(sj >= thresh, jnp.exp(sj - col_max), jnp.float32(0.0))
        w = w / jnp.sum(w, axis=0, keepdims=True)

        idx_j = idx[:, j * _SUB:(j + 1) * _SUB]
        oh = (rowsB == idx_j).astype(jnp.float32)
        rhs = jnp.concatenate([w, ones_rows], axis=0)
        part += lax.dot_general(rhs, oh, (((1,), (1,)), ((), ())),
                                preferred_element_type=jnp.float32)
    acc_ref[...] += part

    @pl.when(i == nb - 1)
    def _fin():
        cnt = acc_ref[T:T + 1, :]
        seg = acc_ref[:T, :] / jnp.maximum(cnt, jnp.float32(1.0))
        out_ref[...] = lax.dot_general(seg, v_ref[...],
                                       (((0,), (0,)), ((), ())),
                                       preferred_element_type=jnp.float32)


def kernel(v_emb, batch_idx, W, b, tokens_K, tokens_V):
    N, D = v_emb.shape
    T = tokens_K.shape[0]
    blk = 2048
    idx = batch_idx.astype(jnp.int32)
    Np = ((N + blk - 1) // blk) * blk
    if Np != N:
        v_emb = jnp.pad(v_emb, ((0, Np - N), (0, 0)))
        idx = jnp.pad(idx, (0, Np - N), constant_values=_B)
    nb = Np // blk
    idx3 = idx.reshape(nb, 1, blk)
    b2 = b.reshape(1, D)

    return pl.pallas_call(
        _body,
        grid=(nb,),
        in_specs=[
            pl.BlockSpec((blk, D), lambda i: (i, 0)),
            pl.BlockSpec((1, 1, blk), lambda i: (i, 0, 0)),
            pl.BlockSpec((D, D), lambda i: (0, 0)),
            pl.BlockSpec((1, D), lambda i: (0, 0)),
            pl.BlockSpec((T, D), lambda i: (0, 0)),
            pl.BlockSpec((T, D), lambda i: (0, 0)),
        ],
        out_specs=pl.BlockSpec((_B, D), lambda i: (0, 0)),
        out_shape=jax.ShapeDtypeStruct((_B, D), jnp.float32),
        scratch_shapes=[pltpu.VMEM((T + 8, _B), jnp.float32)],
    )(v_emb, idx3, W, b2, tokens_K, tokens_V)


# speedup vs baseline: 57.3895x; 1.4395x over previous
import functools
import math

import jax
import jax.numpy as jnp
from jax import lax
from jax.experimental import pallas as pl
from jax.experimental.pallas import tpu as pltpu

_B = 128
_TOPK = 8
_SUB = 256


def _body(x_ref, idx_ref, w_ref, b_ref, k_ref, v_ref, out_ref, acc_ref, *,
          n_valid):
    i = pl.program_id(0)
    nb = pl.num_programs(0)
    blk, D = x_ref.shape
    T = k_ref.shape[0]
    scale = 1.0 / math.sqrt(D)

    @pl.when(i == 0)
    def _init():
        acc_ref[...] = jnp.zeros_like(acc_ref)

    M = lax.dot_general(w_ref[...], k_ref[...], (((0,), (1,)), ((), ())),
                        preferred_element_type=jnp.float32) * scale
    c = lax.dot_general(b_ref[...], k_ref[...], (((1,), (1,)), ((), ())),
                        preferred_element_type=jnp.float32) * scale
    s = jnp.dot(x_ref[...], M, preferred_element_type=jnp.float32) + c
    sT = jnp.swapaxes(s, 0, 1)

    idx = idx_ref[0]
    rowsB = lax.broadcasted_iota(jnp.int32, (_B, _SUB), 0)
    ones_rows = jnp.ones((8, _SUB), jnp.float32)
    neg = jnp.float32(-jnp.inf)

    lane = lax.broadcasted_iota(jnp.int32, (1, _SUB), 1)
    part = jnp.zeros((T + 8, _B), jnp.float32)
    for j in range(blk // _SUB):
        sj = sT[:, j * _SUB:(j + 1) * _SUB]
        valid = (i * blk + j * _SUB + lane) < n_valid
        sj = jnp.where(valid, sj, jnp.float32(0.0))
        cur = sj
        col_max = None
        thresh = None
        for it in range(_TOPK):
            m = jnp.max(cur, axis=0, keepdims=True)
            if it == 0:
                col_max = m
            thresh = m
            if it != _TOPK - 1:
                cur = jnp.where(cur == m, neg, cur)
        w = jnp.where(sj >= thresh, jnp.exp(sj - col_max), jnp.float32(0.0))
        w = w / jnp.sum(w, axis=0, keepdims=True)

        idx_j = idx[:, j * _SUB:(j + 1) * _SUB]
        oh = (rowsB == idx_j).astype(jnp.float32)
        rhs = jnp.concatenate([w, ones_rows], axis=0)
        part += lax.dot_general(rhs, oh, (((1,), (1,)), ((), ())),
                                preferred_element_type=jnp.float32)
    acc_ref[...] += part

    @pl.when(i == nb - 1)
    def _fin():
        cnt = acc_ref[T:T + 1, :]
        seg = acc_ref[:T, :] / jnp.maximum(cnt, jnp.float32(1.0))
        out_ref[...] = lax.dot_general(seg, v_ref[...],
                                       (((0,), (0,)), ((), ())),
                                       preferred_element_type=jnp.float32)


def kernel(v_emb, batch_idx, W, b, tokens_K, tokens_V):
    N, D = v_emb.shape
    T = tokens_K.shape[0]
    blk = 2048
    idx = batch_idx.astype(jnp.int32)
    Np = ((N + blk - 1) // blk) * blk
    if Np != N:
        idx = jnp.pad(idx, (0, Np - N), constant_values=_B)
    nb = Np // blk
    idx3 = idx.reshape(nb, 1, blk)
    b2 = b.reshape(1, D)

    return pl.pallas_call(
        functools.partial(_body, n_valid=N),
        grid=(nb,),
        in_specs=[
            pl.BlockSpec((blk, D), lambda i: (i, 0)),
            pl.BlockSpec((1, 1, blk), lambda i: (i, 0, 0)),
            pl.BlockSpec((D, D), lambda i: (0, 0)),
            pl.BlockSpec((1, D), lambda i: (0, 0)),
            pl.BlockSpec((T, D), lambda i: (0, 0)),
            pl.BlockSpec((T, D), lambda i: (0, 0)),
        ],
        out_specs=pl.BlockSpec((_B, D), lambda i: (0, 0)),
        out_shape=jax.ShapeDtypeStruct((_B, D), jnp.float32),
        scratch_shapes=[pltpu.VMEM((T + 8, _B), jnp.float32)],
    )(v_emb, idx3, W, b2, tokens_K, tokens_V)
